# baseline (speedup 1.0000x reference)
import jax
import jax.numpy as jnp
from jax.experimental import pallas as pl

N = 10000
D = 64
H = 4
COEFF = H * D
NRBF = 20
CUTOFF = 5.0


def _rbf(d):
    mu = jnp.linspace(0.0, CUTOFF, NRBF)
    delta = CUTOFF / (NRBF - 1)
    return jnp.exp(-0.5 / (delta * delta) * (d[:, None] - mu[None, :]) ** 2)


def _scatter_softmax(src, idx, n):
    m = jax.ops.segment_max(src, idx, num_segments=n)
    e = jnp.exp(src - m[idx])
    s = jax.ops.segment_sum(e, idx, num_segments=n)
    return e / s[idx]


def _final_kernel(h_ref, hsem_ref, hsp_ref, x_ref, v_ref, dv_ref, scale_ref,
                  wn1a_ref, wn1b_ref, wn1c_ref, bn1_ref, wn2_ref, bn2_ref,
                  h_out, x_out, v_out):
    h = h_ref[...]
    t = (h @ wn1a_ref[...] + hsem_ref[...] @ wn1b_ref[...]
         + hsp_ref[...] @ wn1c_ref[...] + bn1_ref[...])
    t = t * jax.nn.sigmoid(t)
    t2 = t @ wn2_ref[...] + bn2_ref[...]
    h_out[...] = h + t2 * jax.nn.sigmoid(t2)
    v_upd = scale_ref[...] * v_ref[...] + dv_ref[...]
    v_out[...] = v_upd
    x_out[...] = x_ref[...] + v_upd


def kernel(h, x, v, pairlist, W_in, b_in, W_e1, b_e1, W_e2, b_e2, W_sem, b_sem,
           W_xmix, W_pn1, b_pn1, W_pn2, b_pn2, W_n1, b_n1, W_n2, b_n2,
           W_v1, b_v1, W_v2, W_vmix):
    idx_i = pairlist[0]
    idx_j = pairlist[1]
    n = h.shape[0]
    r_ij = x[idx_j] - x[idx_i]
    d_ij = jnp.sqrt(jnp.sum(r_ij * r_ij, axis=-1) + 1e-12)
    dir_ij = r_ij / (d_ij[:, None] + 1e-05)
    h_cat = jnp.concatenate([h[idx_i], h[idx_j]], axis=-1)
    h_filt = _rbf(d_ij) * (h_cat @ W_in + b_in)
    e_in = jnp.concatenate([h_cat, h_filt, d_ij[:, None]], axis=1)
    h_ij_edge = jax.nn.silu(e_in @ W_e1 + b_e1) @ W_e2 + b_e2
    logits = jax.nn.celu(h_ij_edge @ W_sem + b_sem, alpha=2.0)
    attn = _scatter_softmax(logits, idx_i, n)
    agg = jax.ops.segment_sum(attn, idx_i, num_segments=n)
    normed = attn / agg[idx_i]
    h_ij_sem = jnp.einsum('pf,ph->pfh', h_ij_edge, normed).reshape(-1, COEFF)
    h_i_sem = jax.ops.segment_sum(h_ij_sem, idx_i, num_segments=n)
    mix = jnp.tanh(h_ij_sem @ W_xmix)
    comb = jnp.einsum('px,pc->pcx', dir_ij, mix)
    cnt = jax.ops.segment_sum(jnp.ones_like(d_ij), idx_i, num_segments=n)
    denom = jnp.clip(cnt, 1.0, None)
    comb_mean = jax.ops.segment_sum(comb, idx_i, num_segments=n) / denom[:, None, None]
    norm_sq = jnp.sum(comb_mean ** 2, axis=-1)
    h_i_sp = jax.nn.silu(jax.nn.silu(norm_sq @ W_pn1 + b_pn1) @ W_pn2 + b_pn2)
    v_ij = jnp.squeeze(jnp.swapaxes(comb, -1, -2) @ W_vmix, -1)
    dv = jax.ops.segment_sum(v_ij, idx_i, num_segments=n) / denom[:, None]
    scale = 2.0 * jax.nn.sigmoid(jax.nn.silu(h @ W_v1 + b_v1) @ W_v2)

    Wn1a = W_n1[:D]
    Wn1b = W_n1[D:D + COEFF]
    Wn1c = W_n1[D + COEFF:]
    BN = 1000
    grid = (n // BN,)
    row_bs = lambda w: pl.BlockSpec((BN, w), lambda i: (i, 0))
    full_bs = lambda a, b: pl.BlockSpec((a, b), lambda i: (0, 0))
    h_upd, x_upd, v_upd = pl.pallas_call(
        _final_kernel,
        grid=grid,
        in_specs=[
            row_bs(D), row_bs(COEFF), row_bs(D), row_bs(3), row_bs(3),
            row_bs(3), row_bs(1),
            full_bs(D, D), full_bs(COEFF, D), full_bs(D, D),
            pl.BlockSpec((D,), lambda i: (0,)),
            full_bs(D, D),
            pl.BlockSpec((D,), lambda i: (0,)),
        ],
        out_specs=(row_bs(D), row_bs(3), row_bs(3)),
        out_shape=(
            jax.ShapeDtypeStruct((n, D), jnp.float32),
            jax.ShapeDtypeStruct((n, 3), jnp.float32),
            jax.ShapeDtypeStruct((n, 3), jnp.float32),
        ),
    )(h, h_i_sem, h_i_sp, x, v, dv, scale, Wn1a, Wn1b, Wn1c, b_n1, W_n2, b_n2)
    return (h_upd, x_upd, v_upd)


# speedup vs baseline: 6.4464x; 6.4464x over previous
import jax
import jax.numpy as jnp
from jax import lax
from jax.experimental import pallas as pl
from jax.experimental.pallas import tpu as pltpu
from jax.experimental.pallas import tpu_sc as plsc

N = 10000
E = 160000
D = 64
H = 4
COEFF = H * D
NRBF = 20
CUTOFF = 5.0

NC, NS, NW = 2, 16, 32
EPW = 5120
EP = NW * EPW
CHG = 128
CH3 = 256
CH5 = 64
NPT = 320

def _mesh():
    return plsc.VectorSubcoreMesh(core_axis_name="c", subcore_axis_name="s",
                                  num_cores=NC, num_subcores=NS)


def _wid():
    return lax.axis_index("s") * NC + lax.axis_index("c")


def _sread(ref, i):
    return ref[pl.ds(i, 16)][0]


def _k1_body(tab, ii, jj, gxi, gxj, ibuf, jbuf, rows, sem):
    base0 = _wid() * EPW

    @pl.loop(0, EPW // CHG)
    def _chunk(c):
        base = pl.multiple_of(base0 + c * CHG, 8)
        pltpu.sync_copy(ii.at[pl.ds(base, CHG)], ibuf)
        pltpu.sync_copy(jj.at[pl.ds(base, CHG)], jbuf)
        pltpu.async_copy(tab.at[ibuf], rows, sem).wait()
        pltpu.sync_copy(rows, gxi.at[pl.ds(base, CHG), :])
        pltpu.async_copy(tab.at[jbuf], rows, sem).wait()
        pltpu.sync_copy(rows, gxj.at[pl.ds(base, CHG), :])


def _k1_gather(tab, ii, jj):
    f = pl.kernel(
        _k1_body,
        out_type=[
            jax.ShapeDtypeStruct((EP, 128), jnp.float32),
            jax.ShapeDtypeStruct((EP, 128), jnp.float32),
        ],
        mesh=_mesh(),
        scratch_types=[
            pltpu.VMEM((CHG,), jnp.int32),
            pltpu.VMEM((CHG,), jnp.int32),
            pltpu.VMEM((CHG, 128), jnp.float32),
            pltpu.SemaphoreType.DMA,
        ],
    )
    return f(tab, ii, jj)


def _k2_body(gxi_ref, gxj_ref, win_ref, bin_ref, we1_ref,
             be1_ref, we2_ref, be2_ref, wsem_ref, bsem_ref, ed_ref):
    gxi = gxi_ref[...]
    gxj = gxj_ref[...]
    gi = gxi[:, :D]
    gj = gxj[:, :D]
    r = gxj[:, D:D + 8] - gxi[:, D:D + 8]
    d2 = jnp.sum(r * r, axis=1, keepdims=True)
    d = jnp.sqrt(d2 + 1e-12)
    dir8 = r / (d + 1e-5)

    mu = (lax.broadcasted_iota(jnp.int32, (1, NRBF), 1).astype(jnp.float32)
          * (CUTOFF / (NRBF - 1)))
    delta = CUTOFF / (NRBF - 1)
    rbf = jnp.exp((-0.5 / (delta * delta)) * (d - mu) ** 2)

    win = win_ref[...]
    filt = rbf * (gi @ win[:D] + gj @ win[D:] + bin_ref[...])

    we1 = we1_ref[...]
    t = (gi @ we1[:D] + gj @ we1[D:2 * D] + filt @ we1[2 * D:2 * D + NRBF]
         + d * we1[2 * D + NRBF] + be1_ref[...])
    t = t * jax.nn.sigmoid(t)
    hedge = t @ we2_ref[...] + be2_ref[...]

    lg = hedge @ wsem_ref[...] + bsem_ref[...]
    lg = jnp.maximum(lg, 0.0) + jnp.minimum(2.0 * (jnp.exp(lg * 0.5) - 1.0), 0.0)
    ed_ref[...] = jnp.concatenate(
        [hedge, lg, jnp.zeros((lg.shape[0], 12), jnp.float32), dir8,
         jnp.zeros((lg.shape[0], 128 - D - 16 - 8), jnp.float32)], axis=1)


def _k2_edge(gxi, gxj, W_in, b_in, W_e1, b_e1, W_e2, b_e2, W_sem, b_sem):
    BE = 512
    grid = (EP // BE,)
    row = lambda w: pl.BlockSpec((BE, w), lambda i: (i, 0))
    full2 = lambda a, b: pl.BlockSpec((a, b), lambda i: (0, 0))
    full1 = lambda a: pl.BlockSpec((a,), lambda i: (0,))
    return pl.pallas_call(
        _k2_body,
        grid=grid,
        in_specs=[
            row(128), row(128),
            full2(2 * D, NRBF), full1(NRBF),
            full2(2 * D + NRBF + 1, D), full1(D),
            full2(D, D), full1(D),
            full2(D, H), full1(H),
        ],
        out_specs=row(128),
        out_shape=jax.ShapeDtypeStruct((EP, 128), jnp.float32),
    )(gxi, gxj, W_in, b_in, W_e1, b_e1, W_e2, b_e2, W_sem, b_sem)


def _k3_body(idx, ed, est, normed, ibuf, ebuf, nbuf, estv, marr, sarr, aarr):
    w = _wid()
    pltpu.sync_copy(est, estv)
    e0 = _sread(estv, w)
    e1 = _sread(estv, w + 1)
    nlo = (w * N) >> 5

    @pl.loop(0, NPT)
    def _init(i):
        marr[pl.ds(i * 16, 16)] = jnp.full((16,), -1e30, jnp.float32)
        sarr[pl.ds(i * 16, 16)] = jnp.zeros((16,), jnp.float32)
        aarr[pl.ds(i * 16, 16)] = jnp.zeros((16,), jnp.float32)

    e0a = e0 & ~7
    nch = (e1 - e0a + CH3 - 1) >> 8

    @pl.loop(0, nch)
    def _pa(c):
        base = pl.multiple_of(e0a + c * CH3, 8)
        pltpu.sync_copy(idx.at[pl.ds(base, CH3 + 8)], ibuf.at[pl.ds(0, CH3 + 8)])
        pltpu.sync_copy(ed.at[pl.ds(base, CH3 + 8), :], ebuf)

        @pl.loop(0, CH3)
        def _edge(i):
            e = base + i
            valid = jnp.logical_and(e >= e0, e < e1)

            @pl.when(valid)
            def _():
                loc = (_sread(ibuf, i) - nlo) * 16
                ls = pl.ds(loc, 16)
                lv = ebuf[i, pl.ds(64, 16)]
                m = marr[ls]
                mn = jnp.maximum(m, lv)
                sarr[ls] = sarr[ls] * jnp.exp(m - mn) + jnp.exp(lv - mn)
                marr[ls] = mn

    @pl.loop(0, nch)
    def _pb(c):
        base = pl.multiple_of(e0a + c * CH3, 8)
        pltpu.sync_copy(idx.at[pl.ds(base, CH3 + 8)], ibuf.at[pl.ds(0, CH3 + 8)])
        pltpu.sync_copy(ed.at[pl.ds(base, CH3 + 8), :], ebuf)

        @pl.loop(0, CH3)
        def _edge(i):
            e = base + i
            valid = jnp.logical_and(e >= e0, e < e1)

            @pl.when(valid)
            def _():
                loc = (_sread(ibuf, i) - nlo) * 16
                ls = pl.ds(loc, 16)
                attn = jnp.exp(ebuf[i, pl.ds(64, 16)] - marr[ls]) / sarr[ls]
                aarr[ls] = aarr[ls] + attn

    def _fill(base_w):
        base_r = pl.multiple_of(base_w & ~7, 8)
        ish = base_w & 7
        pltpu.sync_copy(idx.at[pl.ds(base_r, CH3 + 8)],
                        ibuf.at[pl.ds(0, CH3 + 8)])
        pltpu.sync_copy(ed.at[pl.ds(base_r, CH3 + 8), :], ebuf)

        @pl.loop(0, CH3)
        def _edge(i):
            loc = (_sread(ibuf, i + ish) - nlo) * 16
            ls = pl.ds(loc, 16)
            attn = jnp.exp(ebuf[i + ish, pl.ds(64, 16)] - marr[ls]) / sarr[ls]
            nbuf[pl.ds(i * 16, 16)] = attn / aarr[ls]

        pltpu.sync_copy(
            nbuf, normed.at[pl.ds(pl.multiple_of(base_w * 16, 16), CH3 * 16)])

    nfull = (e1 - e0) >> 8

    @pl.loop(0, nfull)
    def _pc(c):
        _fill(e0 + c * CH3)

    nleft = e1 - (e0 + nfull * CH3)

    @pl.when(jnp.logical_and(nleft > 0, e1 - e0 >= CH3))
    def _tail_fast():
        _fill(e1 - CH3)

    @pl.when(jnp.logical_and(nleft > 0, e1 - e0 < CH3))
    def _tail_slow():
        base_r = pl.multiple_of(e0 & ~7, 8)
        ish = e0 & 7
        pltpu.sync_copy(idx.at[pl.ds(base_r, CH3 + 8)],
                        ibuf.at[pl.ds(0, CH3 + 8)])
        pltpu.sync_copy(ed.at[pl.ds(base_r, CH3 + 8), :], ebuf)

        @pl.loop(0, e1 - e0)
        def _edge(i):
            loc = (_sread(ibuf, i + ish) - nlo) * 16
            ls = pl.ds(loc, 16)
            attn = jnp.exp(ebuf[i + ish, pl.ds(64, 16)] - marr[ls]) / sarr[ls]
            nbuf[pl.ds(0, 16)] = attn / aarr[ls]
            pltpu.sync_copy(
                nbuf.at[pl.ds(0, 16)],
                normed.at[pl.ds(pl.multiple_of((e0 + i) * 16, 16), 16)])


def _k3_softmax(idx, ed, est):
    f = pl.kernel(
        _k3_body,
        out_type=[jax.ShapeDtypeStruct((EP * 16,), jnp.float32)],
        mesh=_mesh(),
        scratch_types=[
            pltpu.VMEM((CH3 + 24,), jnp.int32),
            pltpu.VMEM((CH3 + 8, 128), jnp.float32),
            pltpu.VMEM((CH3 * 16,), jnp.float32),
            pltpu.VMEM((48,), jnp.int32),
            pltpu.VMEM((NPT * 16,), jnp.float32),
            pltpu.VMEM((NPT * 16,), jnp.float32),
            pltpu.VMEM((NPT * 16,), jnp.float32),
        ],
    )
    (normed,) = f(idx, ed, est)
    return normed


def _k4_body(ed_ref, nrm_ref, wx_ref, wv_ref,
             hsem_ref, mix_ref, geom_ref):
    BE = ed_ref.shape[0]
    ed = ed_ref[...]
    hedge = ed[:, :D]
    dir8 = ed[:, 80:88]
    rep = lax.broadcasted_iota(jnp.int32, (D, COEFF), 1) // H
    R = jnp.where(rep == lax.broadcasted_iota(jnp.int32, (D, COEFF), 0),
                  1.0, 0.0)
    hm = lax.broadcasted_iota(jnp.int32, (16, COEFF), 1) % H
    S = jnp.where(hm == lax.broadcasted_iota(jnp.int32, (16, COEFF), 0),
                  1.0, 0.0)
    hsem = (hedge @ R) * (nrm_ref[...] @ S)
    hsem_ref[...] = hsem
    mix = jnp.tanh(hsem @ wx_ref[...])
    mix_ref[...] = mix
    wv = mix @ wv_ref[...]
    geom_ref[...] = jnp.concatenate(
        [dir8[:, :3], wv, jnp.zeros((BE, 12), jnp.float32)], axis=1)


def _k4_mix(ed, normed, W_xmix, W_vmix):
    BE = 512
    grid = (EP // BE,)
    row = lambda w: pl.BlockSpec((BE, w), lambda i: (i, 0))
    return pl.pallas_call(
        _k4_body,
        grid=grid,
        in_specs=[
            row(128), row(16),
            pl.BlockSpec((COEFF, COEFF), lambda i: (0, 0)),
            pl.BlockSpec((COEFF, 1), lambda i: (0, 0)),
        ],
        out_specs=(row(COEFF), row(COEFF), row(16)),
        out_shape=(
            jax.ShapeDtypeStruct((EP, COEFF), jnp.float32),
            jax.ShapeDtypeStruct((EP, COEFF), jnp.float32),
            jax.ShapeDtypeStruct((EP, 16), jnp.float32),
        ),
    )(ed, normed, W_xmix, W_vmix)


def _k5_body(idx, hsem, mix, geom, est,
             oh, ocx, ocy, ocz, odv,
             ibuf, hbuf, mbuf, gbuf, estv,
             acc_cx, acc_cy, acc_cz, acc_h, srow, hrow):
    w = _wid()
    pltpu.sync_copy(est, estv)
    e0 = _sread(estv, w)
    e1 = _sread(estv, w + 1)
    nlo = (w * N) >> 5
    nhi = ((w + 1) * N) >> 5

    zero = jnp.zeros((16,), jnp.float32)
    for f in range(16):
        s = pl.ds(16 * f, 16)
        acc_cx[s] = zero
        acc_cy[s] = zero
        acc_cz[s] = zero
        acc_h[s] = zero

    def flush(n, cnt, dvx, dvy, dvz):
        cntf = jnp.maximum(cnt, 1).astype(jnp.float32)
        inv = 1.0 / (jnp.zeros((16,), jnp.float32) + cntf)
        for f in range(16):
            s = pl.ds(16 * f, 16)
            srow[s] = acc_cx[s] * inv
            acc_cx[s] = zero
        pltpu.sync_copy(
            srow, ocx.at[pl.ds(pl.multiple_of(n * COEFF, 8), COEFF)])
        for f in range(16):
            s = pl.ds(16 * f, 16)
            srow[s] = acc_cy[s] * inv
            acc_cy[s] = zero
        pltpu.sync_copy(
            srow, ocy.at[pl.ds(pl.multiple_of(n * COEFF, 8), COEFF)])
        for f in range(16):
            s = pl.ds(16 * f, 16)
            srow[s] = acc_cz[s] * inv
            acc_cz[s] = zero
        pltpu.sync_copy(
            srow, ocz.at[pl.ds(pl.multiple_of(n * COEFF, 8), COEFF)])
        for f in range(16):
            s = pl.ds(16 * f, 16)
            srow[s] = acc_h[s]
            acc_h[s] = zero
        pltpu.sync_copy(
            srow, oh.at[pl.ds(pl.multiple_of(n * COEFF, 8), COEFF)])
        lane = lax.iota(jnp.int32, 16)
        dvrow = (jnp.where(lane == 0, dvx, 0.0)
                 + jnp.where(lane == 1, dvy, 0.0)
                 + jnp.where(lane == 2, dvz, 0.0)) * inv
        hrow[...] = dvrow
        pltpu.sync_copy(
            hrow, odv.at[pl.ds(pl.multiple_of(n * 16, 8), 16)])

    e0a = e0 & ~7
    nch = (e1 - e0a + CH5 - 1) >> 6

    def chunk(c, carry):
        cur, cnt, dvx, dvy, dvz = carry
        base = pl.multiple_of(e0a + c * CH5, 8)
        pltpu.sync_copy(idx.at[pl.ds(base, CH5)], ibuf.at[pl.ds(0, CH5)])
        pltpu.sync_copy(hsem.at[pl.ds(base, CH5), :], hbuf)
        pltpu.sync_copy(mix.at[pl.ds(base, CH5), :], mbuf)
        pltpu.sync_copy(geom.at[pl.ds(base, CH5), :], gbuf)

        def edge(i, icarry):
            cur, cnt, dvx, dvy, dvz = icarry
            e = base + i
            valid = jnp.logical_and(e >= e0, e < e1)

            def do(cur, cnt, dvx, dvy, dvz):
                nd = _sread(ibuf, i)

                def fb(n, fc):
                    c0, dx0, dy0, dz0 = fc
                    flush(n, c0, dx0, dy0, dz0)
                    return (jnp.int32(0), 0.0, 0.0, 0.0)

                cnt, dvx, dvy, dvz = pl.loop(
                    cur, nd, init_carry=(cnt, dvx, dvy, dvz))(fb)
                g = gbuf[i, :]
                dx = g[0]
                dy = g[1]
                dz = g[2]
                wv = g[3]
                for f in range(16):
                    s = pl.ds(16 * f, 16)
                    m = mbuf[i, s]
                    plsc.addupdate(acc_cx.at[s], m * dx)
                    plsc.addupdate(acc_cy.at[s], m * dy)
                    plsc.addupdate(acc_cz.at[s], m * dz)
                    plsc.addupdate(acc_h.at[s], hbuf[i, s])
                return (nd, cnt + 1, dvx + dx * wv, dvy + dy * wv,
                        dvz + dz * wv)

            return lax.cond(valid, do, lambda *a: a, cur, cnt, dvx, dvy, dvz)

        return pl.loop(0, CH5, init_carry=(cur, cnt, dvx, dvy, dvz))(edge)

    cur, cnt, dvx, dvy, dvz = pl.loop(
        0, nch, init_carry=(jnp.int32(nlo), jnp.int32(0), 0.0, 0.0, 0.0))(chunk)

    def fb2(n, fc):
        c0, dx0, dy0, dz0 = fc
        flush(n, c0, dx0, dy0, dz0)
        return (jnp.int32(0), 0.0, 0.0, 0.0)

    pl.loop(cur, nhi, init_carry=(cnt, dvx, dvy, dvz))(fb2)


def _k5_reduce(idx, hsem, mix, geom, est):
    f = pl.kernel(
        _k5_body,
        out_type=[
            jax.ShapeDtypeStruct((N * COEFF,), jnp.float32),
            jax.ShapeDtypeStruct((N * COEFF,), jnp.float32),
            jax.ShapeDtypeStruct((N * COEFF,), jnp.float32),
            jax.ShapeDtypeStruct((N * COEFF,), jnp.float32),
            jax.ShapeDtypeStruct((N * 16,), jnp.float32),
        ],
        mesh=_mesh(),
        scratch_types=[
            pltpu.VMEM((CH5 + 16,), jnp.int32),
            pltpu.VMEM((CH5, COEFF), jnp.float32),
            pltpu.VMEM((CH5, COEFF), jnp.float32),
            pltpu.VMEM((CH5, 16), jnp.float32),
            pltpu.VMEM((48,), jnp.int32),
            pltpu.VMEM((COEFF,), jnp.float32),
            pltpu.VMEM((COEFF,), jnp.float32),
            pltpu.VMEM((COEFF,), jnp.float32),
            pltpu.VMEM((COEFF,), jnp.float32),
            pltpu.VMEM((COEFF,), jnp.float32),
            pltpu.VMEM((16,), jnp.float32),
        ],
    )
    oh, ocx, ocy, ocz, odv = f(idx, hsem, mix, geom, est)
    return oh, ocx, ocy, ocz, odv


def _k6_body(h_ref, hs_ref, cx_ref, cy_ref, cz_ref, dv_ref, x_ref, v_ref,
             wpn1_ref, bpn1_ref, wpn2_ref, bpn2_ref,
             wn1a_ref, wn1b_ref, wn1c_ref, bn1_ref, wn2_ref, bn2_ref,
             wv1_ref, bv1_ref, wv2_ref,
             ho_ref, xo_ref, vo_ref):
    h = h_ref[...]
    cx = cx_ref[...]
    cy = cy_ref[...]
    cz = cz_ref[...]
    nsq = cx * cx + cy * cy + cz * cz
    t = nsq @ wpn1_ref[...] + bpn1_ref[...]
    t = t * jax.nn.sigmoid(t)
    t = t @ wpn2_ref[...] + bpn2_ref[...]
    hsp = t * jax.nn.sigmoid(t)
    u = (h @ wn1a_ref[...] + hs_ref[...] @ wn1b_ref[...]
         + hsp @ wn1c_ref[...] + bn1_ref[...])
    u = u * jax.nn.sigmoid(u)
    u = u @ wn2_ref[...] + bn2_ref[...]
    ho_ref[...] = h + u * jax.nn.sigmoid(u)
    sv = h @ wv1_ref[...] + bv1_ref[...]
    sv = sv * jax.nn.sigmoid(sv)
    scale = 2.0 * jax.nn.sigmoid(sv @ wv2_ref[...])
    vu = scale * v_ref[...] + dv_ref[...][:, :3]
    vo_ref[...] = vu
    xo_ref[...] = x_ref[...] + vu


def _k6_finish(h, hsum, cx, cy, cz, dvm, x, v, W_pn1, b_pn1, W_pn2, b_pn2,
               W_n1, b_n1, W_n2, b_n2, W_v1, b_v1, W_v2):
    BN = 1000
    grid = (N // BN,)
    row = lambda w: pl.BlockSpec((BN, w), lambda i: (i, 0))
    full2 = lambda a, b: pl.BlockSpec((a, b), lambda i: (0, 0))
    full1 = lambda a: pl.BlockSpec((a,), lambda i: (0,))
    Wn1a = W_n1[:D]
    Wn1b = W_n1[D:D + COEFF]
    Wn1c = W_n1[D + COEFF:]
    return pl.pallas_call(
        _k6_body,
        grid=grid,
        in_specs=[
            row(D), row(COEFF), row(COEFF), row(COEFF), row(COEFF),
            row(16), row(3), row(3),
            full2(COEFF, D), full1(D), full2(D, D), full1(D),
            full2(D, D), full2(COEFF, D), full2(D, D), full1(D),
            full2(D, D), full1(D),
            full2(D, D), full1(D), full2(D, 1),
        ],
        out_specs=(row(D), row(3), row(3)),
        out_shape=(
            jax.ShapeDtypeStruct((N, D), jnp.float32),
            jax.ShapeDtypeStruct((N, 3), jnp.float32),
            jax.ShapeDtypeStruct((N, 3), jnp.float32),
        ),
    )(h, hsum, cx, cy, cz, dvm, x, v, W_pn1, b_pn1, W_pn2, b_pn2,
      Wn1a, Wn1b, Wn1c, b_n1, W_n2, b_n2, W_v1, b_v1, W_v2)


def kernel(h, x, v, pairlist, W_in, b_in, W_e1, b_e1, W_e2, b_e2, W_sem,
           b_sem, W_xmix, W_pn1, b_pn1, W_pn2, b_pn2, W_n1, b_n1, W_n2, b_n2,
           W_v1, b_v1, W_v2, W_vmix):
    idx_i = pairlist[0]
    idx_j = pairlist[1]
    ii_s, jj_s = lax.sort([idx_i, idx_j], num_keys=1)
    pad = EP - E
    ii_p = jnp.concatenate([ii_s, jnp.full((pad,), N, jnp.int32)])
    jj_p = jnp.concatenate([jj_s, jnp.full((pad,), N, jnp.int32)])

    nbounds = jnp.array([(t * N) // NW for t in range(NW + 1)] + [0] * 7,
                        jnp.int32)
    est = jnp.searchsorted(ii_p, nbounds[:NW + 1]).astype(jnp.int32)
    est = jnp.concatenate([est, jnp.zeros((48 - (NW + 1),), jnp.int32)])

    tab = jnp.concatenate(
        [jnp.concatenate([h, x, jnp.zeros((N, 128 - D - 3), jnp.float32)],
                         axis=1),
         jnp.zeros((16, 128), jnp.float32)])

    gxi, gxj = _k1_gather(tab, ii_p, jj_p)
    ed = _k2_edge(gxi, gxj, W_in, b_in, W_e1, b_e1, W_e2, b_e2, W_sem, b_sem)
    normed = _k3_softmax(ii_p, ed, est).reshape(EP, 16)
    hsem, mix, geom = _k4_mix(ed, normed, W_xmix, W_vmix)
    hsum, ccx, ccy, ccz, dvm = _k5_reduce(ii_p, hsem, mix, geom, est)
    hsum = hsum.reshape(N, COEFF)
    ccx = ccx.reshape(N, COEFF)
    ccy = ccy.reshape(N, COEFF)
    ccz = ccz.reshape(N, COEFF)
    dvm = dvm.reshape(N, 16)
    return _k6_finish(h, hsum, ccx, ccy, ccz, dvm, x, v, W_pn1, b_pn1,
                      W_pn2, b_pn2, W_n1, b_n1, W_n2, b_n2, W_v1, b_v1, W_v2)


# speedup vs baseline: 6.9207x; 1.0736x over previous
import jax
import jax.numpy as jnp
from jax import lax
from jax.experimental import pallas as pl
from jax.experimental.pallas import tpu as pltpu
from jax.experimental.pallas import tpu_sc as plsc

N = 10000
E = 160000
D = 64
H = 4
COEFF = H * D
NRBF = 20
CUTOFF = 5.0

NC, NS, NW = 2, 16, 32
EPW = 5120
EP = NW * EPW
CHG = 128
CH3 = 256
CH5 = 128
NPT = 320

def _mesh():
    return plsc.VectorSubcoreMesh(core_axis_name="c", subcore_axis_name="s",
                                  num_cores=NC, num_subcores=NS)


def _wid():
    return lax.axis_index("s") * NC + lax.axis_index("c")


def _sread(ref, i):
    return ref[pl.ds(i, 16)][0]


def _k1_body(tab, ii, jj, gxi, gxj, ibuf, jbuf, rowsi, rowsj, gsi, gsj, osem):
    base0 = _wid() * EPW

    @pl.loop(0, EPW // CHG)
    def _chunk(c):
        p = c & 1
        base = pl.multiple_of(base0 + c * CHG, 8)
        pltpu.sync_copy(ii.at[pl.ds(base, CHG)], ibuf)
        pltpu.sync_copy(jj.at[pl.ds(base, CHG)], jbuf)

        @pl.when(c >= 2)
        def _drain():
            pltpu.make_async_copy(
                rowsi.at[p], gxi.at[pl.ds(0, CHG), :], osem).wait()
            pltpu.make_async_copy(
                rowsj.at[p], gxj.at[pl.ds(0, CHG), :], osem).wait()

        di = pltpu.async_copy(tab.at[ibuf], rowsi.at[p], gsi)
        dj = pltpu.async_copy(tab.at[jbuf], rowsj.at[p], gsj)
        di.wait()
        dj.wait()
        pltpu.async_copy(rowsi.at[p], gxi.at[pl.ds(base, CHG), :], osem)
        pltpu.async_copy(rowsj.at[p], gxj.at[pl.ds(base, CHG), :], osem)

    @pl.loop(0, 2)
    def _final_drain(i):
        pltpu.make_async_copy(
            rowsi.at[0], gxi.at[pl.ds(0, CHG), :], osem).wait()
        pltpu.make_async_copy(
            rowsj.at[0], gxj.at[pl.ds(0, CHG), :], osem).wait()


def _k1_gather(tab, ii, jj):
    f = pl.kernel(
        _k1_body,
        out_type=[
            jax.ShapeDtypeStruct((EP, 128), jnp.float32),
            jax.ShapeDtypeStruct((EP, 128), jnp.float32),
        ],
        mesh=_mesh(),
        scratch_types=[
            pltpu.VMEM((CHG,), jnp.int32),
            pltpu.VMEM((CHG,), jnp.int32),
            pltpu.VMEM((2, CHG, 128), jnp.float32),
            pltpu.VMEM((2, CHG, 128), jnp.float32),
            pltpu.SemaphoreType.DMA,
            pltpu.SemaphoreType.DMA,
            pltpu.SemaphoreType.DMA,
        ],
    )
    return f(tab, ii, jj)


def _k2_body(gxi_ref, gxj_ref, win_ref, bin_ref, we1_ref,
             be1_ref, we2_ref, be2_ref, wsem_ref, bsem_ref, ed_ref):
    gxi = gxi_ref[...]
    gxj = gxj_ref[...]
    gi = gxi[:, :D]
    gj = gxj[:, :D]
    r = gxj[:, D:D + 8] - gxi[:, D:D + 8]
    d2 = jnp.sum(r * r, axis=1, keepdims=True)
    d = jnp.sqrt(d2 + 1e-12)
    dir8 = r / (d + 1e-5)

    mu = (lax.broadcasted_iota(jnp.int32, (1, NRBF), 1).astype(jnp.float32)
          * (CUTOFF / (NRBF - 1)))
    delta = CUTOFF / (NRBF - 1)
    rbf = jnp.exp((-0.5 / (delta * delta)) * (d - mu) ** 2)

    win = win_ref[...]
    filt = rbf * (gi @ win[:D] + gj @ win[D:] + bin_ref[...])

    we1 = we1_ref[...]
    t = (gi @ we1[:D] + gj @ we1[D:2 * D] + filt @ we1[2 * D:2 * D + NRBF]
         + d * we1[2 * D + NRBF] + be1_ref[...])
    t = t * jax.nn.sigmoid(t)
    hedge = t @ we2_ref[...] + be2_ref[...]

    lg = hedge @ wsem_ref[...] + bsem_ref[...]
    lg = jnp.maximum(lg, 0.0) + jnp.minimum(2.0 * (jnp.exp(lg * 0.5) - 1.0), 0.0)
    ed_ref[...] = jnp.concatenate(
        [hedge, lg, jnp.zeros((lg.shape[0], 12), jnp.float32), dir8,
         jnp.zeros((lg.shape[0], 128 - D - 16 - 8), jnp.float32)], axis=1)


def _k2_edge(gxi, gxj, W_in, b_in, W_e1, b_e1, W_e2, b_e2, W_sem, b_sem):
    BE = 512
    grid = (EP // BE,)
    row = lambda w: pl.BlockSpec((BE, w), lambda i: (i, 0))
    full2 = lambda a, b: pl.BlockSpec((a, b), lambda i: (0, 0))
    full1 = lambda a: pl.BlockSpec((a,), lambda i: (0,))
    return pl.pallas_call(
        _k2_body,
        grid=grid,
        in_specs=[
            row(128), row(128),
            full2(2 * D, NRBF), full1(NRBF),
            full2(2 * D + NRBF + 1, D), full1(D),
            full2(D, D), full1(D),
            full2(D, H), full1(H),
        ],
        out_specs=row(128),
        out_shape=jax.ShapeDtypeStruct((EP, 128), jnp.float32),
    )(gxi, gxj, W_in, b_in, W_e1, b_e1, W_e2, b_e2, W_sem, b_sem)


def _k3_body(idx, ed, est, normed, ibuf, ebuf, nbuf, estv, marr, sarr):
    w = _wid()
    pltpu.sync_copy(est, estv)
    e0 = _sread(estv, w)
    e1 = _sread(estv, w + 1)
    nlo = (w * N) >> 5

    @pl.loop(0, NPT)
    def _init(i):
        marr[pl.ds(i * 16, 16)] = jnp.full((16,), -1e30, jnp.float32)
        sarr[pl.ds(i * 16, 16)] = jnp.zeros((16,), jnp.float32)

    e0a = e0 & ~7
    nch = (e1 - e0a + CH3 - 1) >> 8

    @pl.loop(0, nch)
    def _pa(c):
        base = pl.multiple_of(e0a + c * CH3, 8)
        pltpu.sync_copy(idx.at[pl.ds(base, CH3 + 8)], ibuf.at[pl.ds(0, CH3 + 8)])
        pltpu.sync_copy(ed.at[pl.ds(base, CH3 + 8), :], ebuf)

        @pl.loop(0, CH3)
        def _edge(i):
            e = base + i
            valid = jnp.logical_and(e >= e0, e < e1)

            @pl.when(valid)
            def _():
                loc = (_sread(ibuf, i) - nlo) * 16
                ls = pl.ds(loc, 16)
                lv = ebuf[i, pl.ds(64, 16)]
                m = marr[ls]
                mn = jnp.maximum(m, lv)
                sarr[ls] = sarr[ls] * jnp.exp(m - mn) + jnp.exp(lv - mn)
                marr[ls] = mn

    def _fill(base_w):
        base_r = pl.multiple_of(base_w & ~7, 8)
        ish = base_w & 7
        pltpu.sync_copy(idx.at[pl.ds(base_r, CH3 + 8)],
                        ibuf.at[pl.ds(0, CH3 + 8)])
        pltpu.sync_copy(ed.at[pl.ds(base_r, CH3 + 8), :], ebuf)

        @pl.loop(0, CH3)
        def _edge(i):
            loc = (_sread(ibuf, i + ish) - nlo) * 16
            ls = pl.ds(loc, 16)
            nbuf[pl.ds(i * 16, 16)] = (
                jnp.exp(ebuf[i + ish, pl.ds(64, 16)] - marr[ls]) / sarr[ls])

        pltpu.sync_copy(
            nbuf, normed.at[pl.ds(pl.multiple_of(base_w * 16, 16), CH3 * 16)])

    nfull = (e1 - e0) >> 8

    @pl.loop(0, nfull)
    def _pc(c):
        _fill(e0 + c * CH3)

    nleft = e1 - (e0 + nfull * CH3)

    @pl.when(jnp.logical_and(nleft > 0, e1 - e0 >= CH3))
    def _tail_fast():
        _fill(e1 - CH3)

    @pl.when(jnp.logical_and(nleft > 0, e1 - e0 < CH3))
    def _tail_slow():
        base_r = pl.multiple_of(e0 & ~7, 8)
        ish = e0 & 7
        pltpu.sync_copy(idx.at[pl.ds(base_r, CH3 + 8)],
                        ibuf.at[pl.ds(0, CH3 + 8)])
        pltpu.sync_copy(ed.at[pl.ds(base_r, CH3 + 8), :], ebuf)

        @pl.loop(0, e1 - e0)
        def _edge(i):
            loc = (_sread(ibuf, i + ish) - nlo) * 16
            ls = pl.ds(loc, 16)
            nbuf[pl.ds(0, 16)] = (
                jnp.exp(ebuf[i + ish, pl.ds(64, 16)] - marr[ls]) / sarr[ls])
            pltpu.sync_copy(
                nbuf.at[pl.ds(0, 16)],
                normed.at[pl.ds(pl.multiple_of((e0 + i) * 16, 16), 16)])


def _k3_softmax(idx, ed, est):
    f = pl.kernel(
        _k3_body,
        out_type=[jax.ShapeDtypeStruct((EP * 16,), jnp.float32)],
        mesh=_mesh(),
        scratch_types=[
            pltpu.VMEM((CH3 + 24,), jnp.int32),
            pltpu.VMEM((CH3 + 8, 128), jnp.float32),
            pltpu.VMEM((CH3 * 16,), jnp.float32),
            pltpu.VMEM((48,), jnp.int32),
            pltpu.VMEM((NPT * 16,), jnp.float32),
            pltpu.VMEM((NPT * 16,), jnp.float32),
        ],
    )
    (normed,) = f(idx, ed, est)
    return normed


def _k4_body(ed_ref, nrm_ref, wx_ref, wv_ref,
             hsem_ref, mix_ref, geom_ref):
    BE = ed_ref.shape[0]
    ed = ed_ref[...]
    hedge = ed[:, :D]
    dir8 = ed[:, 80:88]
    rep = lax.broadcasted_iota(jnp.int32, (D, COEFF), 1) // H
    R = jnp.where(rep == lax.broadcasted_iota(jnp.int32, (D, COEFF), 0),
                  1.0, 0.0)
    hm = lax.broadcasted_iota(jnp.int32, (16, COEFF), 1) % H
    S = jnp.where(hm == lax.broadcasted_iota(jnp.int32, (16, COEFF), 0),
                  1.0, 0.0)
    hsem = (hedge @ R) * (nrm_ref[...] @ S)
    hsem_ref[...] = hsem
    mix = jnp.tanh(hsem @ wx_ref[...])
    mix_ref[...] = mix
    wv = mix @ wv_ref[...]
    geom_ref[...] = jnp.concatenate(
        [dir8[:, :3], wv, jnp.zeros((BE, 12), jnp.float32)], axis=1)


def _k4_mix(ed, normed, W_xmix, W_vmix):
    BE = 512
    grid = (EP // BE,)
    row = lambda w: pl.BlockSpec((BE, w), lambda i: (i, 0))
    return pl.pallas_call(
        _k4_body,
        grid=grid,
        in_specs=[
            row(128), row(16),
            pl.BlockSpec((COEFF, COEFF), lambda i: (0, 0)),
            pl.BlockSpec((COEFF, 1), lambda i: (0, 0)),
        ],
        out_specs=(row(COEFF), row(COEFF), row(16)),
        out_shape=(
            jax.ShapeDtypeStruct((EP, COEFF), jnp.float32),
            jax.ShapeDtypeStruct((EP, COEFF), jnp.float32),
            jax.ShapeDtypeStruct((EP, 16), jnp.float32),
        ),
    )(ed, normed, W_xmix, W_vmix)


ROW = 1088

def _k5_body(idx, hsem, mix, geom, est, oall,
             ibuf, hbuf, mbuf, gbuf, estv,
             acc_cx, acc_cy, acc_cz, acc_h, stage, sem):
    w = _wid()
    pltpu.sync_copy(est, estv)
    e0 = _sread(estv, w)
    e1 = _sread(estv, w + 1)
    nlo = (w * N) >> 5
    nhi = ((w + 1) * N) >> 5

    zero = jnp.zeros((16,), jnp.float32)
    for f in range(16):
        s = pl.ds(16 * f, 16)
        acc_cx[s] = zero
        acc_cy[s] = zero
        acc_cz[s] = zero
        acc_h[s] = zero

    def flush(n, cnt, dvx, dvy, dvz):
        k = n - nlo

        @pl.when(k >= 4)
        def _drain():
            pltpu.make_async_copy(
                stage.at[pl.ds(0, ROW)], oall.at[pl.ds(0, ROW)], sem).wait()

        so = (k & 3) * ROW
        cntf = jnp.maximum(cnt, 1).astype(jnp.float32)
        inv = 1.0 / (jnp.zeros((16,), jnp.float32) + cntf)
        for f in range(16):
            s = pl.ds(16 * f, 16)
            stage[pl.ds(so + 16 * f, 16)] = acc_cx[s] * inv
            acc_cx[s] = zero
        for f in range(16):
            s = pl.ds(16 * f, 16)
            stage[pl.ds(so + 256 + 16 * f, 16)] = acc_cy[s] * inv
            acc_cy[s] = zero
        for f in range(16):
            s = pl.ds(16 * f, 16)
            stage[pl.ds(so + 512 + 16 * f, 16)] = acc_cz[s] * inv
            acc_cz[s] = zero
        for f in range(16):
            s = pl.ds(16 * f, 16)
            stage[pl.ds(so + 768 + 16 * f, 16)] = acc_h[s]
            acc_h[s] = zero
        lane = lax.iota(jnp.int32, 16)
        dvrow = (jnp.where(lane == 0, dvx, 0.0)
                 + jnp.where(lane == 1, dvy, 0.0)
                 + jnp.where(lane == 2, dvz, 0.0)) * inv
        stage[pl.ds(so + 1024, 16)] = dvrow
        stage[pl.ds(so + 1040, 16)] = zero
        stage[pl.ds(so + 1056, 16)] = zero
        stage[pl.ds(so + 1072, 16)] = zero
        pltpu.async_copy(
            stage.at[pl.ds(pl.multiple_of(so, 8), ROW)],
            oall.at[pl.ds(pl.multiple_of(n * ROW, 8), ROW)], sem)

    e0a = e0 & ~7
    nch = (e1 - e0a + CH5 - 1) >> 7

    def chunk(c, carry):
        cur, cnt, dvx, dvy, dvz = carry
        base = pl.multiple_of(e0a + c * CH5, 8)
        pltpu.sync_copy(idx.at[pl.ds(base, CH5)], ibuf.at[pl.ds(0, CH5)])
        pltpu.sync_copy(hsem.at[pl.ds(base, CH5), :], hbuf)
        pltpu.sync_copy(mix.at[pl.ds(base, CH5), :], mbuf)
        pltpu.sync_copy(geom.at[pl.ds(base, CH5), :], gbuf)

        def edge(i, icarry):
            cur, cnt, dvx, dvy, dvz = icarry
            e = base + i
            valid = jnp.logical_and(e >= e0, e < e1)

            def do(cur, cnt, dvx, dvy, dvz):
                nd = _sread(ibuf, i)

                def fb(n, fc):
                    c0, dx0, dy0, dz0 = fc
                    flush(n, c0, dx0, dy0, dz0)
                    return (jnp.int32(0), 0.0, 0.0, 0.0)

                cnt, dvx, dvy, dvz = pl.loop(
                    cur, nd, init_carry=(cnt, dvx, dvy, dvz))(fb)
                g = gbuf[i, :]
                dx = g[0]
                dy = g[1]
                dz = g[2]
                wv = g[3]
                for f in range(16):
                    s = pl.ds(16 * f, 16)
                    m = mbuf[i, s]
                    plsc.addupdate(acc_cx.at[s], m * dx)
                    plsc.addupdate(acc_cy.at[s], m * dy)
                    plsc.addupdate(acc_cz.at[s], m * dz)
                    plsc.addupdate(acc_h.at[s], hbuf[i, s])
                return (nd, cnt + 1, dvx + dx * wv, dvy + dy * wv,
                        dvz + dz * wv)

            return lax.cond(valid, do, lambda *a: a, cur, cnt, dvx, dvy, dvz)

        return pl.loop(0, CH5, init_carry=(cur, cnt, dvx, dvy, dvz))(edge)

    cur, cnt, dvx, dvy, dvz = pl.loop(
        0, nch, init_carry=(jnp.int32(nlo), jnp.int32(0), 0.0, 0.0, 0.0))(chunk)

    def fb2(n, fc):
        c0, dx0, dy0, dz0 = fc
        flush(n, c0, dx0, dy0, dz0)
        return (jnp.int32(0), 0.0, 0.0, 0.0)

    pl.loop(cur, nhi, init_carry=(cnt, dvx, dvy, dvz))(fb2)

    @pl.loop(0, 4)
    def _final_drain(i):
        pltpu.make_async_copy(
            stage.at[pl.ds(0, ROW)], oall.at[pl.ds(0, ROW)], sem).wait()


def _k5_reduce(idx, hsem, mix, geom, est):
    f = pl.kernel(
        _k5_body,
        out_type=[jax.ShapeDtypeStruct((N * ROW,), jnp.float32)],
        mesh=_mesh(),
        scratch_types=[
            pltpu.VMEM((CH5 + 16,), jnp.int32),
            pltpu.VMEM((CH5, COEFF), jnp.float32),
            pltpu.VMEM((CH5, COEFF), jnp.float32),
            pltpu.VMEM((CH5, 16), jnp.float32),
            pltpu.VMEM((48,), jnp.int32),
            pltpu.VMEM((COEFF,), jnp.float32),
            pltpu.VMEM((COEFF,), jnp.float32),
            pltpu.VMEM((COEFF,), jnp.float32),
            pltpu.VMEM((COEFF,), jnp.float32),
            pltpu.VMEM((4 * ROW,), jnp.float32),
            pltpu.SemaphoreType.DMA,
        ],
    )
    (oall,) = f(idx, hsem, mix, geom, est)
    return oall


def _k6_body(h_ref, oall_ref, x_ref, v_ref,
             wpn1_ref, bpn1_ref, wpn2_ref, bpn2_ref,
             wn1a_ref, wn1b_ref, wn1c_ref, bn1_ref, wn2_ref, bn2_ref,
             wv1_ref, bv1_ref, wv2_ref,
             ho_ref, xo_ref, vo_ref):
    h = h_ref[...]
    oall = oall_ref[...]
    cx = oall[:, :COEFF]
    cy = oall[:, COEFF:2 * COEFF]
    cz = oall[:, 2 * COEFF:3 * COEFF]
    hs = oall[:, 3 * COEFF:4 * COEFF]
    dv3 = oall[:, 4 * COEFF:4 * COEFF + 3]
    nsq = cx * cx + cy * cy + cz * cz
    t = nsq @ wpn1_ref[...] + bpn1_ref[...]
    t = t * jax.nn.sigmoid(t)
    t = t @ wpn2_ref[...] + bpn2_ref[...]
    hsp = t * jax.nn.sigmoid(t)
    u = (h @ wn1a_ref[...] + hs @ wn1b_ref[...]
         + hsp @ wn1c_ref[...] + bn1_ref[...])
    u = u * jax.nn.sigmoid(u)
    u = u @ wn2_ref[...] + bn2_ref[...]
    ho_ref[...] = h + u * jax.nn.sigmoid(u)
    sv = h @ wv1_ref[...] + bv1_ref[...]
    sv = sv * jax.nn.sigmoid(sv)
    scale = 2.0 * jax.nn.sigmoid(sv @ wv2_ref[...])
    vu = scale * v_ref[...] + dv3
    vo_ref[...] = vu
    xo_ref[...] = x_ref[...] + vu


def _k6_finish(h, oall, x, v, W_pn1, b_pn1, W_pn2, b_pn2,
               W_n1, b_n1, W_n2, b_n2, W_v1, b_v1, W_v2):
    BN = 1000
    grid = (N // BN,)
    row = lambda w: pl.BlockSpec((BN, w), lambda i: (i, 0))
    full2 = lambda a, b: pl.BlockSpec((a, b), lambda i: (0, 0))
    full1 = lambda a: pl.BlockSpec((a,), lambda i: (0,))
    Wn1a = W_n1[:D]
    Wn1b = W_n1[D:D + COEFF]
    Wn1c = W_n1[D + COEFF:]
    return pl.pallas_call(
        _k6_body,
        grid=grid,
        in_specs=[
            row(D), row(ROW), row(3), row(3),
            full2(COEFF, D), full1(D), full2(D, D), full1(D),
            full2(D, D), full2(COEFF, D), full2(D, D), full1(D),
            full2(D, D), full1(D),
            full2(D, D), full1(D), full2(D, 1),
        ],
        out_specs=(row(D), row(3), row(3)),
        out_shape=(
            jax.ShapeDtypeStruct((N, D), jnp.float32),
            jax.ShapeDtypeStruct((N, 3), jnp.float32),
            jax.ShapeDtypeStruct((N, 3), jnp.float32),
        ),
    )(h, oall, x, v, W_pn1, b_pn1, W_pn2, b_pn2,
      Wn1a, Wn1b, Wn1c, b_n1, W_n2, b_n2, W_v1, b_v1, W_v2)


def kernel(h, x, v, pairlist, W_in, b_in, W_e1, b_e1, W_e2, b_e2, W_sem,
           b_sem, W_xmix, W_pn1, b_pn1, W_pn2, b_pn2, W_n1, b_n1, W_n2, b_n2,
           W_v1, b_v1, W_v2, W_vmix):
    idx_i = pairlist[0]
    idx_j = pairlist[1]
    ii_s, jj_s = lax.sort([idx_i, idx_j], num_keys=1)
    pad = EP - E
    ii_p = jnp.concatenate([ii_s, jnp.full((pad,), N, jnp.int32)])
    jj_p = jnp.concatenate([jj_s, jnp.full((pad,), N, jnp.int32)])

    nbounds = jnp.array([(t * N) // NW for t in range(NW + 1)] + [0] * 7,
                        jnp.int32)
    est = jnp.searchsorted(ii_p, nbounds[:NW + 1]).astype(jnp.int32)
    est = jnp.concatenate([est, jnp.zeros((48 - (NW + 1),), jnp.int32)])

    tab = jnp.concatenate(
        [jnp.concatenate([h, x, jnp.zeros((N, 128 - D - 3), jnp.float32)],
                         axis=1),
         jnp.zeros((16, 128), jnp.float32)])

    gxi, gxj = _k1_gather(tab, ii_p, jj_p)
    ed = _k2_edge(gxi, gxj, W_in, b_in, W_e1, b_e1, W_e2, b_e2, W_sem, b_sem)
    normed = _k3_softmax(ii_p, ed, est).reshape(EP, 16)
    hsem, mix, geom = _k4_mix(ed, normed, W_xmix, W_vmix)
    oall = _k5_reduce(ii_p, hsem, mix, geom, est).reshape(N, ROW)
    return _k6_finish(h, oall, x, v, W_pn1, b_pn1,
                      W_pn2, b_pn2, W_n1, b_n1, W_n2, b_n2, W_v1, b_v1, W_v2)


# speedup vs baseline: 7.3061x; 1.0557x over previous
import jax
import jax.numpy as jnp
from jax import lax
from jax.experimental import pallas as pl
from jax.experimental.pallas import tpu as pltpu
from jax.experimental.pallas import tpu_sc as plsc

N = 10000
E = 160000
D = 64
H = 4
COEFF = H * D
NRBF = 20
CUTOFF = 5.0

NC, NS, NW = 2, 16, 32
EPW = 5120
EP = NW * EPW
CHG = 128
CH3 = 256
CH5 = 64
NPT = 320

def _mesh():
    return plsc.VectorSubcoreMesh(core_axis_name="c", subcore_axis_name="s",
                                  num_cores=NC, num_subcores=NS)


def _wid():
    return lax.axis_index("s") * NC + lax.axis_index("c")


def _sread(ref, i):
    return ref[pl.ds(i, 16)][0]


def _k1_body(tab, ii, jj, gxi, gxj, ibuf, jbuf, rowsi, rowsj, gsi, gsj, osem):
    base0 = _wid() * EPW

    @pl.loop(0, EPW // CHG)
    def _chunk(c):
        p = c & 1
        base = pl.multiple_of(base0 + c * CHG, 8)
        pltpu.sync_copy(ii.at[pl.ds(base, CHG)], ibuf)
        pltpu.sync_copy(jj.at[pl.ds(base, CHG)], jbuf)

        @pl.when(c >= 2)
        def _drain():
            pltpu.make_async_copy(
                rowsi.at[p], gxi.at[pl.ds(0, CHG), :], osem).wait()
            pltpu.make_async_copy(
                rowsj.at[p], gxj.at[pl.ds(0, CHG), :], osem).wait()

        di = pltpu.async_copy(tab.at[ibuf], rowsi.at[p], gsi)
        dj = pltpu.async_copy(tab.at[jbuf], rowsj.at[p], gsj)
        di.wait()
        dj.wait()
        pltpu.async_copy(rowsi.at[p], gxi.at[pl.ds(base, CHG), :], osem)
        pltpu.async_copy(rowsj.at[p], gxj.at[pl.ds(base, CHG), :], osem)

    @pl.loop(0, 2)
    def _final_drain(i):
        pltpu.make_async_copy(
            rowsi.at[0], gxi.at[pl.ds(0, CHG), :], osem).wait()
        pltpu.make_async_copy(
            rowsj.at[0], gxj.at[pl.ds(0, CHG), :], osem).wait()


def _k1_gather(tab, ii, jj):
    f = pl.kernel(
        _k1_body,
        out_type=[
            jax.ShapeDtypeStruct((EP, 128), jnp.float32),
            jax.ShapeDtypeStruct((EP, 128), jnp.float32),
        ],
        mesh=_mesh(),
        scratch_types=[
            pltpu.VMEM((CHG,), jnp.int32),
            pltpu.VMEM((CHG,), jnp.int32),
            pltpu.VMEM((2, CHG, 128), jnp.float32),
            pltpu.VMEM((2, CHG, 128), jnp.float32),
            pltpu.SemaphoreType.DMA,
            pltpu.SemaphoreType.DMA,
            pltpu.SemaphoreType.DMA,
        ],
    )
    return f(tab, ii, jj)


def _k2_body(gxi_ref, gxj_ref, win_ref, bin_ref, we1_ref,
             be1_ref, we2_ref, be2_ref, wsem_ref, bsem_ref, ed_ref):
    gxi = gxi_ref[...]
    gxj = gxj_ref[...]
    gi = gxi[:, :D]
    gj = gxj[:, :D]
    r = gxj[:, D:D + 8] - gxi[:, D:D + 8]
    d2 = jnp.sum(r * r, axis=1, keepdims=True)
    d = jnp.sqrt(d2 + 1e-12)
    dir8 = r / (d + 1e-5)

    mu = (lax.broadcasted_iota(jnp.int32, (1, NRBF), 1).astype(jnp.float32)
          * (CUTOFF / (NRBF - 1)))
    delta = CUTOFF / (NRBF - 1)
    rbf = jnp.exp((-0.5 / (delta * delta)) * (d - mu) ** 2)

    win = win_ref[...]
    filt = rbf * (gi @ win[:D] + gj @ win[D:] + bin_ref[...])

    we1 = we1_ref[...]
    t = (gi @ we1[:D] + gj @ we1[D:2 * D] + filt @ we1[2 * D:2 * D + NRBF]
         + d * we1[2 * D + NRBF] + be1_ref[...])
    t = t * jax.nn.sigmoid(t)
    hedge = t @ we2_ref[...] + be2_ref[...]

    lg = hedge @ wsem_ref[...] + bsem_ref[...]
    lg = jnp.maximum(lg, 0.0) + jnp.minimum(2.0 * (jnp.exp(lg * 0.5) - 1.0), 0.0)
    ed_ref[...] = jnp.concatenate(
        [hedge, lg, jnp.zeros((lg.shape[0], 12), jnp.float32), dir8,
         jnp.zeros((lg.shape[0], 128 - D - 16 - 8), jnp.float32)], axis=1)


def _k2_edge(gxi, gxj, W_in, b_in, W_e1, b_e1, W_e2, b_e2, W_sem, b_sem):
    BE = 512
    grid = (EP // BE,)
    row = lambda w: pl.BlockSpec((BE, w), lambda i: (i, 0))
    full2 = lambda a, b: pl.BlockSpec((a, b), lambda i: (0, 0))
    full1 = lambda a: pl.BlockSpec((a,), lambda i: (0,))
    return pl.pallas_call(
        _k2_body,
        grid=grid,
        in_specs=[
            row(128), row(128),
            full2(2 * D, NRBF), full1(NRBF),
            full2(2 * D + NRBF + 1, D), full1(D),
            full2(D, D), full1(D),
            full2(D, H), full1(H),
        ],
        out_specs=row(128),
        out_shape=jax.ShapeDtypeStruct((EP, 128), jnp.float32),
    )(gxi, gxj, W_in, b_in, W_e1, b_e1, W_e2, b_e2, W_sem, b_sem)


def _k3_body(idx, ed, est, normed, ibuf, ebuf, nbuf, estv, marr, sarr):
    w = _wid()
    pltpu.sync_copy(est, estv)
    e0 = _sread(estv, w)
    e1 = _sread(estv, w + 1)
    nlo = (w * N) >> 5

    @pl.loop(0, NPT)
    def _init(i):
        marr[pl.ds(i * 16, 16)] = jnp.full((16,), -1e30, jnp.float32)
        sarr[pl.ds(i * 16, 16)] = jnp.zeros((16,), jnp.float32)

    e0a = e0 & ~7
    nch = (e1 - e0a + CH3 - 1) >> 8

    @pl.loop(0, nch)
    def _pa(c):
        base = pl.multiple_of(e0a + c * CH3, 8)
        pltpu.sync_copy(idx.at[pl.ds(base, CH3 + 8)], ibuf.at[pl.ds(0, CH3 + 8)])
        pltpu.sync_copy(ed.at[pl.ds(base, CH3 + 8), :], ebuf)

        @pl.loop(0, CH3)
        def _edge(i):
            e = base + i
            valid = jnp.logical_and(e >= e0, e < e1)

            @pl.when(valid)
            def _():
                loc = (_sread(ibuf, i) - nlo) * 16
                ls = pl.ds(loc, 16)
                lv = ebuf[i, pl.ds(64, 16)]
                m = marr[ls]
                mn = jnp.maximum(m, lv)
                sarr[ls] = sarr[ls] * jnp.exp(m - mn) + jnp.exp(lv - mn)
                marr[ls] = mn

    def _fill(base_w):
        base_r = pl.multiple_of(base_w & ~7, 8)
        ish = base_w & 7
        pltpu.sync_copy(idx.at[pl.ds(base_r, CH3 + 8)],
                        ibuf.at[pl.ds(0, CH3 + 8)])
        pltpu.sync_copy(ed.at[pl.ds(base_r, CH3 + 8), :], ebuf)

        @pl.loop(0, CH3)
        def _edge(i):
            loc = (_sread(ibuf, i + ish) - nlo) * 16
            ls = pl.ds(loc, 16)
            nbuf[pl.ds(i * 16, 16)] = (
                jnp.exp(ebuf[i + ish, pl.ds(64, 16)] - marr[ls]) / sarr[ls])

        pltpu.sync_copy(
            nbuf, normed.at[pl.ds(pl.multiple_of(base_w * 16, 16), CH3 * 16)])

    nfull = (e1 - e0) >> 8

    @pl.loop(0, nfull)
    def _pc(c):
        _fill(e0 + c * CH3)

    nleft = e1 - (e0 + nfull * CH3)

    @pl.when(jnp.logical_and(nleft > 0, e1 - e0 >= CH3))
    def _tail_fast():
        _fill(e1 - CH3)

    @pl.when(jnp.logical_and(nleft > 0, e1 - e0 < CH3))
    def _tail_slow():
        base_r = pl.multiple_of(e0 & ~7, 8)
        ish = e0 & 7
        pltpu.sync_copy(idx.at[pl.ds(base_r, CH3 + 8)],
                        ibuf.at[pl.ds(0, CH3 + 8)])
        pltpu.sync_copy(ed.at[pl.ds(base_r, CH3 + 8), :], ebuf)

        @pl.loop(0, e1 - e0)
        def _edge(i):
            loc = (_sread(ibuf, i + ish) - nlo) * 16
            ls = pl.ds(loc, 16)
            nbuf[pl.ds(0, 16)] = (
                jnp.exp(ebuf[i + ish, pl.ds(64, 16)] - marr[ls]) / sarr[ls])
            pltpu.sync_copy(
                nbuf.at[pl.ds(0, 16)],
                normed.at[pl.ds(pl.multiple_of((e0 + i) * 16, 16), 16)])


def _k3_softmax(idx, ed, est):
    f = pl.kernel(
        _k3_body,
        out_type=[jax.ShapeDtypeStruct((EP * 16,), jnp.float32)],
        mesh=_mesh(),
        scratch_types=[
            pltpu.VMEM((CH3 + 24,), jnp.int32),
            pltpu.VMEM((CH3 + 8, 128), jnp.float32),
            pltpu.VMEM((CH3 * 16,), jnp.float32),
            pltpu.VMEM((48,), jnp.int32),
            pltpu.VMEM((NPT * 16,), jnp.float32),
            pltpu.VMEM((NPT * 16,), jnp.float32),
        ],
    )
    (normed,) = f(idx, ed, est)
    return normed


def _k4_body(ed_ref, nrm_ref, wx_ref, wv_ref,
             hsem_ref, mix_ref, geom_ref):
    BE = ed_ref.shape[0]
    ed = ed_ref[...]
    hedge = ed[:, :D]
    dir8 = ed[:, 80:88]
    rep = lax.broadcasted_iota(jnp.int32, (D, COEFF), 1) // H
    R = jnp.where(rep == lax.broadcasted_iota(jnp.int32, (D, COEFF), 0),
                  1.0, 0.0)
    hm = lax.broadcasted_iota(jnp.int32, (16, COEFF), 1) % H
    S = jnp.where(hm == lax.broadcasted_iota(jnp.int32, (16, COEFF), 0),
                  1.0, 0.0)
    hsem = (hedge @ R) * (nrm_ref[...] @ S)
    hsem_ref[...] = hsem
    mix = jnp.tanh(hsem @ wx_ref[...])
    mix_ref[...] = mix
    wv = mix @ wv_ref[...]
    geom_ref[...] = jnp.concatenate(
        [dir8[:, :3], wv, jnp.zeros((BE, 12), jnp.float32)], axis=1)


def _k4_mix(ed, normed, W_xmix, W_vmix):
    BE = 512
    grid = (EP // BE,)
    row = lambda w: pl.BlockSpec((BE, w), lambda i: (i, 0))
    return pl.pallas_call(
        _k4_body,
        grid=grid,
        in_specs=[
            row(128), row(16),
            pl.BlockSpec((COEFF, COEFF), lambda i: (0, 0)),
            pl.BlockSpec((COEFF, 1), lambda i: (0, 0)),
        ],
        out_specs=(row(COEFF), row(COEFF), row(16)),
        out_shape=(
            jax.ShapeDtypeStruct((EP, COEFF), jnp.float32),
            jax.ShapeDtypeStruct((EP, COEFF), jnp.float32),
            jax.ShapeDtypeStruct((EP, 16), jnp.float32),
        ),
    )(ed, normed, W_xmix, W_vmix)


ROW = 1088

def _k5_body(idx, hsem, mix, geom, est, oall,
             ibuf, hbuf, mbuf, gbuf, estv,
             acc_cx, acc_cy, acc_cz, acc_h, stage, sem, lsem):
    w = _wid()
    pltpu.sync_copy(est, estv)
    e0 = _sread(estv, w)
    e1 = _sread(estv, w + 1)
    nlo = (w * N) >> 5
    nhi = ((w + 1) * N) >> 5

    zero = jnp.zeros((16,), jnp.float32)
    for f in range(16):
        s = pl.ds(16 * f, 16)
        acc_cx[s] = zero
        acc_cy[s] = zero
        acc_cz[s] = zero
        acc_h[s] = zero

    def flush(n, cnt, dvx, dvy, dvz):
        k = n - nlo

        @pl.when(k >= 4)
        def _drain():
            pltpu.make_async_copy(
                stage.at[pl.ds(0, ROW)], oall.at[pl.ds(0, ROW)], sem).wait()

        so = (k & 3) * ROW
        cntf = jnp.maximum(cnt, 1).astype(jnp.float32)
        inv = 1.0 / (jnp.zeros((16,), jnp.float32) + cntf)
        for f in range(16):
            s = pl.ds(16 * f, 16)
            stage[pl.ds(so + 16 * f, 16)] = acc_cx[s] * inv
            acc_cx[s] = zero
        for f in range(16):
            s = pl.ds(16 * f, 16)
            stage[pl.ds(so + 256 + 16 * f, 16)] = acc_cy[s] * inv
            acc_cy[s] = zero
        for f in range(16):
            s = pl.ds(16 * f, 16)
            stage[pl.ds(so + 512 + 16 * f, 16)] = acc_cz[s] * inv
            acc_cz[s] = zero
        for f in range(16):
            s = pl.ds(16 * f, 16)
            stage[pl.ds(so + 768 + 16 * f, 16)] = acc_h[s]
            acc_h[s] = zero
        lane = lax.iota(jnp.int32, 16)
        dvrow = (jnp.where(lane == 0, dvx, 0.0)
                 + jnp.where(lane == 1, dvy, 0.0)
                 + jnp.where(lane == 2, dvz, 0.0)) * inv
        stage[pl.ds(so + 1024, 16)] = dvrow
        stage[pl.ds(so + 1040, 16)] = zero
        stage[pl.ds(so + 1056, 16)] = zero
        stage[pl.ds(so + 1072, 16)] = zero
        pltpu.async_copy(
            stage.at[pl.ds(pl.multiple_of(so, 8), ROW)],
            oall.at[pl.ds(pl.multiple_of(n * ROW, 8), ROW)], sem)

    e0a = e0 & ~7
    nch = (e1 - e0a + CH5 - 1) >> 6

    def start_load(c):
        p = c & 1
        base = pl.multiple_of(e0a + c * CH5, 8)
        pltpu.async_copy(hsem.at[pl.ds(base, CH5), :], hbuf.at[p], lsem)
        pltpu.async_copy(mix.at[pl.ds(base, CH5), :], mbuf.at[p], lsem)
        pltpu.async_copy(geom.at[pl.ds(base, CH5), :], gbuf.at[p], lsem)

    @pl.when(nch > 0)
    def _prime():
        start_load(0)

    def chunk(c, carry):
        cur, cnt, dvx, dvy, dvz = carry
        p = c & 1
        base = pl.multiple_of(e0a + c * CH5, 8)
        pltpu.sync_copy(idx.at[pl.ds(base, CH5)], ibuf.at[pl.ds(0, CH5)])
        pltpu.make_async_copy(
            hsem.at[pl.ds(0, CH5), :], hbuf.at[p], lsem).wait()
        pltpu.make_async_copy(
            mix.at[pl.ds(0, CH5), :], mbuf.at[p], lsem).wait()
        pltpu.make_async_copy(
            geom.at[pl.ds(0, CH5), :], gbuf.at[p], lsem).wait()

        @pl.when(c + 1 < nch)
        def _next():
            start_load(c + 1)

        def edge(i, icarry):
            cur, cnt, dvx, dvy, dvz = icarry
            e = base + i
            valid = jnp.logical_and(e >= e0, e < e1)

            def do(cur, cnt, dvx, dvy, dvz):
                nd = _sread(ibuf, i)

                def fb(n, fc):
                    c0, dx0, dy0, dz0 = fc
                    flush(n, c0, dx0, dy0, dz0)
                    return (jnp.int32(0), 0.0, 0.0, 0.0)

                cnt, dvx, dvy, dvz = pl.loop(
                    cur, nd, init_carry=(cnt, dvx, dvy, dvz))(fb)
                g = gbuf[p, i, :]
                dx = g[0]
                dy = g[1]
                dz = g[2]
                wv = g[3]
                for f in range(16):
                    s = pl.ds(16 * f, 16)
                    m = mbuf[p, i, s]
                    plsc.addupdate(acc_cx.at[s], m * dx)
                    plsc.addupdate(acc_cy.at[s], m * dy)
                    plsc.addupdate(acc_cz.at[s], m * dz)
                    plsc.addupdate(acc_h.at[s], hbuf[p, i, s])
                return (nd, cnt + 1, dvx + dx * wv, dvy + dy * wv,
                        dvz + dz * wv)

            return lax.cond(valid, do, lambda *a: a, cur, cnt, dvx, dvy, dvz)

        return pl.loop(0, CH5, init_carry=(cur, cnt, dvx, dvy, dvz))(edge)

    cur, cnt, dvx, dvy, dvz = pl.loop(
        0, nch, init_carry=(jnp.int32(nlo), jnp.int32(0), 0.0, 0.0, 0.0))(chunk)

    def fb2(n, fc):
        c0, dx0, dy0, dz0 = fc
        flush(n, c0, dx0, dy0, dz0)
        return (jnp.int32(0), 0.0, 0.0, 0.0)

    pl.loop(cur, nhi, init_carry=(cnt, dvx, dvy, dvz))(fb2)

    @pl.loop(0, 4)
    def _final_drain(i):
        pltpu.make_async_copy(
            stage.at[pl.ds(0, ROW)], oall.at[pl.ds(0, ROW)], sem).wait()


def _k5_reduce(idx, hsem, mix, geom, est):
    f = pl.kernel(
        _k5_body,
        out_type=[jax.ShapeDtypeStruct((N * ROW,), jnp.float32)],
        mesh=_mesh(),
        scratch_types=[
            pltpu.VMEM((CH5 + 16,), jnp.int32),
            pltpu.VMEM((2, CH5, COEFF), jnp.float32),
            pltpu.VMEM((2, CH5, COEFF), jnp.float32),
            pltpu.VMEM((2, CH5, 16), jnp.float32),
            pltpu.VMEM((48,), jnp.int32),
            pltpu.VMEM((COEFF,), jnp.float32),
            pltpu.VMEM((COEFF,), jnp.float32),
            pltpu.VMEM((COEFF,), jnp.float32),
            pltpu.VMEM((COEFF,), jnp.float32),
            pltpu.VMEM((4 * ROW,), jnp.float32),
            pltpu.SemaphoreType.DMA,
            pltpu.SemaphoreType.DMA,
        ],
    )
    (oall,) = f(idx, hsem, mix, geom, est)
    return oall


def _k6_body(h_ref, oall_ref, x_ref, v_ref,
             wpn1_ref, bpn1_ref, wpn2_ref, bpn2_ref,
             wn1a_ref, wn1b_ref, wn1c_ref, bn1_ref, wn2_ref, bn2_ref,
             wv1_ref, bv1_ref, wv2_ref,
             ho_ref, xo_ref, vo_ref):
    h = h_ref[...]
    oall = oall_ref[...]
    cx = oall[:, :COEFF]
    cy = oall[:, COEFF:2 * COEFF]
    cz = oall[:, 2 * COEFF:3 * COEFF]
    hs = oall[:, 3 * COEFF:4 * COEFF]
    dv3 = oall[:, 4 * COEFF:4 * COEFF + 3]
    nsq = cx * cx + cy * cy + cz * cz
    t = nsq @ wpn1_ref[...] + bpn1_ref[...]
    t = t * jax.nn.sigmoid(t)
    t = t @ wpn2_ref[...] + bpn2_ref[...]
    hsp = t * jax.nn.sigmoid(t)
    u = (h @ wn1a_ref[...] + hs @ wn1b_ref[...]
         + hsp @ wn1c_ref[...] + bn1_ref[...])
    u = u * jax.nn.sigmoid(u)
    u = u @ wn2_ref[...] + bn2_ref[...]
    ho_ref[...] = h + u * jax.nn.sigmoid(u)
    sv = h @ wv1_ref[...] + bv1_ref[...]
    sv = sv * jax.nn.sigmoid(sv)
    scale = 2.0 * jax.nn.sigmoid(sv @ wv2_ref[...])
    vu = scale * v_ref[...] + dv3
    vo_ref[...] = vu
    xo_ref[...] = x_ref[...] + vu


def _k6_finish(h, oall, x, v, W_pn1, b_pn1, W_pn2, b_pn2,
               W_n1, b_n1, W_n2, b_n2, W_v1, b_v1, W_v2):
    BN = 1000
    grid = (N // BN,)
    row = lambda w: pl.BlockSpec((BN, w), lambda i: (i, 0))
    full2 = lambda a, b: pl.BlockSpec((a, b), lambda i: (0, 0))
    full1 = lambda a: pl.BlockSpec((a,), lambda i: (0,))
    Wn1a = W_n1[:D]
    Wn1b = W_n1[D:D + COEFF]
    Wn1c = W_n1[D + COEFF:]
    return pl.pallas_call(
        _k6_body,
        grid=grid,
        in_specs=[
            row(D), row(ROW), row(3), row(3),
            full2(COEFF, D), full1(D), full2(D, D), full1(D),
            full2(D, D), full2(COEFF, D), full2(D, D), full1(D),
            full2(D, D), full1(D),
            full2(D, D), full1(D), full2(D, 1),
        ],
        out_specs=(row(D), row(3), row(3)),
        out_shape=(
            jax.ShapeDtypeStruct((N, D), jnp.float32),
            jax.ShapeDtypeStruct((N, 3), jnp.float32),
            jax.ShapeDtypeStruct((N, 3), jnp.float32),
        ),
    )(h, oall, x, v, W_pn1, b_pn1, W_pn2, b_pn2,
      Wn1a, Wn1b, Wn1c, b_n1, W_n2, b_n2, W_v1, b_v1, W_v2)


def kernel(h, x, v, pairlist, W_in, b_in, W_e1, b_e1, W_e2, b_e2, W_sem,
           b_sem, W_xmix, W_pn1, b_pn1, W_pn2, b_pn2, W_n1, b_n1, W_n2, b_n2,
           W_v1, b_v1, W_v2, W_vmix):
    idx_i = pairlist[0]
    idx_j = pairlist[1]
    ii_s, jj_s = lax.sort([idx_i, idx_j], num_keys=1)
    pad = EP - E
    ii_p = jnp.concatenate([ii_s, jnp.full((pad,), N, jnp.int32)])
    jj_p = jnp.concatenate([jj_s, jnp.full((pad,), N, jnp.int32)])

    nbounds = jnp.array([(t * N) // NW for t in range(NW + 1)] + [0] * 7,
                        jnp.int32)
    est = jnp.searchsorted(ii_p, nbounds[:NW + 1]).astype(jnp.int32)
    est = jnp.concatenate([est, jnp.zeros((48 - (NW + 1),), jnp.int32)])

    tab = jnp.concatenate(
        [jnp.concatenate([h, x, jnp.zeros((N, 128 - D - 3), jnp.float32)],
                         axis=1),
         jnp.zeros((16, 128), jnp.float32)])

    gxi, gxj = _k1_gather(tab, ii_p, jj_p)
    ed = _k2_edge(gxi, gxj, W_in, b_in, W_e1, b_e1, W_e2, b_e2, W_sem, b_sem)
    normed = _k3_softmax(ii_p, ed, est).reshape(EP, 16)
    hsem, mix, geom = _k4_mix(ed, normed, W_xmix, W_vmix)
    oall = _k5_reduce(ii_p, hsem, mix, geom, est).reshape(N, ROW)
    return _k6_finish(h, oall, x, v, W_pn1, b_pn1,
                      W_pn2, b_pn2, W_n1, b_n1, W_n2, b_n2, W_v1, b_v1, W_v2)
